# baseline (speedup 1.0000x reference)
import functools

import jax
import jax.numpy as jnp
from jax import lax
from jax.experimental import pallas as pl
from jax.experimental.pallas import tpu as pltpu
from jax.experimental.pallas import tpu_sc as plsc

_N = 10000
_E = 320000
_F = 128
_B = 16
_K = 3
_R = 10000

_NW = 32
_EPW = _E // _NW
_C = 80
_DEN_R = (_N + 127) // 128
_DEN_RP = 80
_ACC_R = _N + _DEN_RP


def _prep_body(ent_ref, q_ref, batch_ref, rel_ref, wq_ref, ws1_ref, wd1_ref,
               wr1_ref, wr2_ref, x_ref, hs1_ref, hd1_ref, relp1_ref, relp2_ref):
    qw = jnp.dot(q_ref[...], wq_ref[...], preferred_element_type=jnp.float32)
    onehot = (batch_ref[...] == lax.broadcasted_iota(
        jnp.int32, (batch_ref.shape[0], _B), 1)).astype(jnp.float32)
    x = ent_ref[...] + jnp.dot(onehot, qw, preferred_element_type=jnp.float32)
    x_ref[...] = x
    hs1_ref[...] = jnp.dot(x, ws1_ref[...], preferred_element_type=jnp.float32)
    hd1_ref[...] = jnp.dot(x, wd1_ref[...], preferred_element_type=jnp.float32)
    r = rel_ref[...]
    relp1_ref[...] = jnp.dot(r, wr1_ref[...], preferred_element_type=jnp.float32)
    relp2_ref[...] = jnp.dot(r, wr2_ref[...], preferred_element_type=jnp.float32)


def _tc_prep(entities, queries, batch2d, relations, W_q, Ws1, Wd1, Wr1, Wr2):
    blk = 1000
    grid = _N // blk
    row_spec = pl.BlockSpec((blk, _F), lambda i: (i, 0))
    full = lambda s: pl.BlockSpec(s, lambda i: tuple(0 for _ in s))
    return pl.pallas_call(
        _prep_body,
        grid=(grid,),
        in_specs=[
            row_spec,
            full((_B, _F)),
            pl.BlockSpec((blk, 1), lambda i: (i, 0)),
            row_spec,
            full((_F, _F)), full((_F, _F)), full((_F, _F)),
            full((_F, _F)), full((_F, _F)),
        ],
        out_specs=[row_spec, row_spec, row_spec, row_spec, row_spec],
        out_shape=[jax.ShapeDtypeStruct((_N, _F), jnp.float32)] * 3
        + [jax.ShapeDtypeStruct((_R, _F), jnp.float32)] * 2,
    )(entities, queries, batch2d, relations, W_q, Ws1, Wd1, Wr1, Wr2)


def _sc_edge_body(hs_hbm, hd_hbm, relp_hbm, src_hbm, dst_hbm, rel_hbm, a_hbm,
                  z_hbm, out_hbm, src_v, dst_v, rel_v, drow_v, dmod_v, didx_v,
                  hs_r, hd_r, relp_r, den_l, a_v, acc, s0, s1, s2):
    cid = lax.axis_index("c")
    sid = lax.axis_index("s")
    wid = cid * 16 + sid

    @pl.when(sid == 0)
    def _():
        pltpu.sync_copy(z_hbm, acc)

    pltpu.sync_copy(a_hbm, a_v)
    plsc.subcore_barrier()

    lanes = lax.broadcasted_iota(jnp.int32, (16,), 0)
    mask0 = lanes == 0

    zf = jnp.zeros((16,), jnp.float32)
    for t in range(_DEN_RP // 16):
        didx_v[pl.ds(16 * t, 16)] = lanes + (16 * t + _N)

    def zero_body(r, carry):
        for j in range(8):
            den_l[r, pl.ds(16 * j, 16)] = zf
        return carry

    lax.fori_loop(0, _DEN_RP, zero_body, 0)

    def chunk_body(ci, carry):
        e0 = wid * _EPW + ci * _C
        pltpu.sync_copy(src_hbm.at[pl.ds(e0, _C)], src_v)
        pltpu.sync_copy(dst_hbm.at[pl.ds(e0, _C)], dst_v)
        pltpu.sync_copy(rel_hbm.at[pl.ds(e0, _C)], rel_v)
        cp1 = pltpu.async_copy(hs_hbm.at[src_v], hs_r, s0)
        cp2 = pltpu.async_copy(hd_hbm.at[dst_v], hd_r, s1)
        cp3 = pltpu.async_copy(relp_hbm.at[rel_v], relp_r, s2)
        for t in range(_C // 16):
            sl = pl.ds(16 * t, 16)
            d = dst_v[sl]
            drow_v[sl] = jnp.right_shift(d, 7)
            dmod_v[sl] = jnp.bitwise_and(d, 127)
        cp1.wait()
        cp2.wait()
        cp3.wait()

        def edge_body(e, carry2):
            sacc = jnp.zeros((16,), jnp.float32)
            hsum = []
            for j in range(8):
                sl = pl.ds(16 * j, 16)
                hs_j = hs_r[e, sl] + relp_r[e, sl]
                z_j = hs_j + hd_r[e, sl]
                lr_j = jnp.maximum(z_j, 0.2 * z_j)
                sacc = sacc + lr_j * a_v[sl]
                hsum.append(hs_j)
            for shift in (1, 2, 4, 8):
                perm = jnp.bitwise_xor(lanes, shift)
                sacc = sacc + sacc.at[perm].get(mode="promise_in_bounds")
            ex = jnp.exp(sacc)
            for j in range(8):
                sl = pl.ds(16 * j, 16)
                hs_r[e, sl] = ex * hsum[j]
            row_w = drow_v[pl.ds(e, 16)]
            col_w = dmod_v[pl.ds(e, 16)]
            plsc.addupdate_scatter(den_l, [row_w, col_w], ex, mask=mask0)
            return carry2

        lax.fori_loop(0, _C, edge_body, 0)
        pltpu.sync_copy(hs_r, acc.at[dst_v], add=True)
        return carry

    lax.fori_loop(0, _EPW // _C, chunk_body, 0)
    pltpu.sync_copy(den_l, acc.at[didx_v], add=True)
    plsc.subcore_barrier()

    @pl.when(sid == 0)
    def _():
        pltpu.sync_copy(acc, out_hbm.at[cid])


def _sc_edge(hs, hd, relp, src, dst, rel, a_vec, zeros_acc):
    mesh = plsc.VectorSubcoreMesh(core_axis_name="c", subcore_axis_name="s")
    kern = functools.partial(
        pl.kernel,
        mesh=mesh,
        compiler_params=pltpu.CompilerParams(needs_layout_passes=False),
        out_type=jax.ShapeDtypeStruct((2, _ACC_R, _F), jnp.float32),
        scratch_types=[
            pltpu.VMEM((_C,), jnp.int32),
            pltpu.VMEM((_C,), jnp.int32),
            pltpu.VMEM((_C,), jnp.int32),
            pltpu.VMEM((_C + 16,), jnp.int32),
            pltpu.VMEM((_C + 16,), jnp.int32),
            pltpu.VMEM((_DEN_RP,), jnp.int32),
            pltpu.VMEM((_C, _F), jnp.float32),
            pltpu.VMEM((_C, _F), jnp.float32),
            pltpu.VMEM((_C, _F), jnp.float32),
            pltpu.VMEM((_DEN_RP, _F), jnp.float32),
            pltpu.VMEM((_F,), jnp.float32),
            pltpu.VMEM_SHARED((_ACC_R, _F), jnp.float32),
            pltpu.SemaphoreType.DMA,
            pltpu.SemaphoreType.DMA,
            pltpu.SemaphoreType.DMA,
        ],
    )(_sc_edge_body)
    return kern(hs, hd, relp, src, dst, rel, a_vec, zeros_acc)


def _mid_body(x_ref, numa_ref, numb_ref, dena_ref, denb_ref, ws_ref, wd_ref,
              inj_ref, hs_ref, hd_ref):
    num = numa_ref[...] + numb_ref[...]
    den = dena_ref[...] + denb_ref[...]
    inj = x_ref[...] + num / (den + 1e-16)
    inj_ref[...] = inj
    hs_ref[...] = jnp.dot(inj, ws_ref[...], preferred_element_type=jnp.float32)
    hd_ref[...] = jnp.dot(inj, wd_ref[...], preferred_element_type=jnp.float32)


def _tc_mid(x, num_a, num_b, den_a, den_b, Ws2, Wd2):
    blk = 1000
    grid = _N // blk
    row_spec = pl.BlockSpec((blk, _F), lambda i: (i, 0))
    den_spec = pl.BlockSpec((blk, 1), lambda i: (i, 0))
    full = lambda s: pl.BlockSpec(s, lambda i: tuple(0 for _ in s))
    return pl.pallas_call(
        _mid_body,
        grid=(grid,),
        in_specs=[row_spec, row_spec, row_spec, den_spec, den_spec,
                  full((_F, _F)), full((_F, _F))],
        out_specs=[row_spec, row_spec, row_spec],
        out_shape=[jax.ShapeDtypeStruct((_N, _F), jnp.float32)] * 3,
    )(x, num_a, num_b, den_a, den_b, Ws2, Wd2)


def _final_body(inj_ref, numa_ref, numb_ref, dena_ref, denb_ref, batch_ref,
                wg_ref, bg_ref, out_ref):
    num = numa_ref[...] + numb_ref[...]
    den = dena_ref[...] + denb_ref[...]
    emb = inj_ref[...] + num / (den + 1e-16)
    gate = jnp.dot(emb, wg_ref[...], preferred_element_type=jnp.float32) \
        + bg_ref[...]
    onehot = (batch_ref[...] == lax.broadcasted_iota(
        jnp.int32, (_N, _B), 1)).astype(jnp.float32)
    ohb = onehot > 0.5
    ms = []
    for k in range(_K):
        gk = gate[:, k:k + 1]
        mk = jnp.max(jnp.where(ohb, gk, -1e30), axis=0, keepdims=True)
        ms.append(mk)
    m_all = jnp.concatenate(ms, axis=0)
    m_node = lax.dot_general(onehot, m_all,
                             (((1,), (1,)), ((), ())),
                             preferred_element_type=jnp.float32)
    ex = jnp.exp(gate - m_node)
    den_b = lax.dot_general(onehot, ex,
                            (((0,), (0,)), ((), ())),
                            preferred_element_type=jnp.float32)
    den_node = jnp.dot(onehot, den_b, preferred_element_type=jnp.float32)
    alpha = ex / (den_node + 1e-16)
    for k in range(_K):
        w = onehot * alpha[:, k:k + 1]
        out_k = lax.dot_general(w, emb,
                                (((0,), (0,)), ((), ())),
                                preferred_element_type=jnp.float32)
        out_ref[:, k, :] = out_k


def _tc_final(inj, num_a, num_b, den_a, den_b, batch2d, W_gate, b_gate2d):
    full = lambda s: pl.BlockSpec(s, lambda: tuple(0 for _ in s))
    return pl.pallas_call(
        _final_body,
        in_specs=[full((_N, _F)), full((_N, _F)), full((_N, _F)),
                  full((_N, 1)), full((_N, 1)),
                  full((_N, 1)), full((_F, _K)), full((1, _K))],
        out_specs=full((_B, _K, _F)),
        out_shape=jax.ShapeDtypeStruct((_B, _K, _F), jnp.float32),
    )(inj, num_a, num_b, den_a, den_b, batch2d, W_gate, b_gate2d)


def kernel(queries, entities, relations, x_coo, batch, W_q, W_inj_src,
           W_inj_dst, a_inj, W_rel_inj, W_enc_src, W_enc_dst, a_enc,
           W_rel_enc, W_gate, b_gate):
    src = x_coo[:, 0]
    rel = x_coo[:, 1]
    dst = x_coo[:, 2]
    batch2d = batch.reshape(_N, 1)
    zeros_acc = jnp.zeros((_ACC_R, _F), jnp.float32)
    b_gate2d = b_gate.reshape(1, _K)

    def split_acc(acc):
        num_a = acc[0, :_N, :]
        num_b = acc[1, :_N, :]
        den_a = acc[0, _N:_N + _DEN_R, :].reshape(_DEN_R * _F)[:_N].reshape(_N, 1)
        den_b = acc[1, _N:_N + _DEN_R, :].reshape(_DEN_R * _F)[:_N].reshape(_N, 1)
        return num_a, num_b, den_a, den_b

    x, hs1, hd1, relp1, relp2 = _tc_prep(
        entities, queries, batch2d, relations,
        W_q, W_inj_src, W_inj_dst, W_rel_inj, W_rel_enc)

    acc1 = _sc_edge(hs1, hd1, relp1, src, dst, rel, a_inj, zeros_acc)

    inj, hs2, hd2 = _tc_mid(x, *split_acc(acc1), W_enc_src, W_enc_dst)

    acc2 = _sc_edge(hs2, hd2, relp2, src, dst, rel, a_enc, zeros_acc)

    return _tc_final(inj, *split_acc(acc2), batch2d, W_gate, b_gate2d)


# speedup vs baseline: 1.2730x; 1.2730x over previous
import functools

import jax
import jax.numpy as jnp
from jax import lax
from jax.experimental import pallas as pl
from jax.experimental.pallas import tpu as pltpu
from jax.experimental.pallas import tpu_sc as plsc

_N = 10000
_E = 320000
_F = 128
_B = 16
_K = 3
_R = 10000

_NW = 32
_EPW = _E // _NW
_C = 40
_CD = 48
_NCH = _EPW // _C
_NG = _NCH // 2
_DEN_R = (_N + 127) // 128
_DEN_RP = 80
_ACC_R = _N + _DEN_RP


def _prep_body(ent_ref, q_ref, batch_ref, rel_ref, wq_ref, ws1_ref, wd1_ref,
               wr1_ref, wr2_ref, x_ref, hs1_ref, hd1_ref, relp1_ref, relp2_ref):
    qw = jnp.dot(q_ref[...], wq_ref[...], preferred_element_type=jnp.float32)
    onehot = (batch_ref[...] == lax.broadcasted_iota(
        jnp.int32, (batch_ref.shape[0], _B), 1)).astype(jnp.float32)
    x = ent_ref[...] + jnp.dot(onehot, qw, preferred_element_type=jnp.float32)
    x_ref[...] = x
    hs1_ref[...] = jnp.dot(x, ws1_ref[...], preferred_element_type=jnp.float32)
    hd1_ref[...] = jnp.dot(x, wd1_ref[...], preferred_element_type=jnp.float32)
    r = rel_ref[...]
    relp1_ref[...] = jnp.dot(r, wr1_ref[...], preferred_element_type=jnp.float32)
    relp2_ref[...] = jnp.dot(r, wr2_ref[...], preferred_element_type=jnp.float32)


def _tc_prep(entities, queries, batch2d, relations, W_q, Ws1, Wd1, Wr1, Wr2):
    blk = 1000
    grid = _N // blk
    row_spec = pl.BlockSpec((blk, _F), lambda i: (i, 0))
    full = lambda s: pl.BlockSpec(s, lambda i: tuple(0 for _ in s))
    return pl.pallas_call(
        _prep_body,
        grid=(grid,),
        in_specs=[
            row_spec,
            full((_B, _F)),
            pl.BlockSpec((blk, 1), lambda i: (i, 0)),
            row_spec,
            full((_F, _F)), full((_F, _F)), full((_F, _F)),
            full((_F, _F)), full((_F, _F)),
        ],
        out_specs=[row_spec, row_spec, row_spec, row_spec, row_spec],
        out_shape=[jax.ShapeDtypeStruct((_N, _F), jnp.float32)] * 3
        + [jax.ShapeDtypeStruct((_R, _F), jnp.float32)] * 2,
    )(entities, queries, batch2d, relations, W_q, Ws1, Wd1, Wr1, Wr2)


def _sc_edge_body(hs_hbm, hd_hbm, relp_hbm, idx_hbm, a_hbm, z_hbm, out_hbm,
                  idx_v0, idx_v1, dst_v0, dst_v1, dblk_v0, dblk_v1, dmod_v0,
                  dmod_v1, hs_r0, hs_r1, hd_r0, hd_r1, relp_r0, relp_r1, a_v,
                  acc, g0, g1, i0, i1, sc0, sc1):
    cid = lax.axis_index("c")
    sid = lax.axis_index("s")
    wid = cid * 16 + sid

    idx_v = (idx_v0, idx_v1)
    dst_v = (dst_v0, dst_v1)
    dblk_v = (dblk_v0, dblk_v1)
    dmod_v = (dmod_v0, dmod_v1)
    hs_r = (hs_r0, hs_r1)
    hd_r = (hd_r0, hd_r1)
    relp_r = (relp_r0, relp_r1)
    gsem = (g0, g1)
    isem = (i0, i1)
    ssem = (sc0, sc1)

    @pl.when(sid == 0)
    def _():
        pltpu.sync_copy(z_hbm, acc)

    pltpu.sync_copy(a_hbm, a_v)
    plsc.subcore_barrier()

    lanes = lax.broadcasted_iota(jnp.int32, (16,), 0)
    zf = jnp.zeros((16,), jnp.float32)

    for b in (0, 1):
        for r in range(_C, _CD):
            for j in range(8):
                hs_r[b][r, pl.ds(16 * j, 16)] = zf
                hd_r[b][r, pl.ds(16 * j, 16)] = zf

    def issue_gathers(b, ci):
        pltpu.async_copy(hs_hbm.at[idx_v[b].at[pl.ds(0, _C)]],
                         hs_r[b].at[pl.ds(0, _C)], gsem[b])
        pltpu.async_copy(hd_hbm.at[idx_v[b].at[pl.ds(_C, _C)]],
                         hd_r[b].at[pl.ds(0, _C)], gsem[b])
        pltpu.async_copy(relp_hbm.at[idx_v[b].at[pl.ds(2 * _C, _C)]],
                         relp_r[b], gsem[b])

    def wait_gathers(b):
        pltpu.make_async_copy(hs_hbm.at[idx_v[b].at[pl.ds(0, _C)]],
                              hs_r[b].at[pl.ds(0, _C)], gsem[b]).wait()
        pltpu.make_async_copy(hd_hbm.at[idx_v[b].at[pl.ds(_C, _C)]],
                              hd_r[b].at[pl.ds(0, _C)], gsem[b]).wait()
        pltpu.make_async_copy(relp_hbm.at[idx_v[b].at[pl.ds(2 * _C, _C)]],
                              relp_r[b], gsem[b]).wait()

    def issue_idx(b, row):
        pltpu.async_copy(idx_hbm.at[row], idx_v[b], isem[b])

    def wait_idx(b, row):
        pltpu.make_async_copy(idx_hbm.at[row], idx_v[b], isem[b]).wait()

    def issue_scatters(b):
        pltpu.async_copy(hs_r[b], acc.at[dst_v[b]], ssem[b], add=True)
        pltpu.async_copy(hd_r[b], acc.at[dblk_v[b]], ssem[b], add=True)

    def wait_scatters(b):
        pltpu.make_async_copy(hs_r[b], acc.at[dst_v[b]], ssem[b]).wait()
        pltpu.make_async_copy(hd_r[b], acc.at[dblk_v[b]], ssem[b]).wait()

    def compute_chunk(b):
        for t in range(_CD // 16):
            sl = pl.ds(16 * t, 16)
            d = idx_v[b][pl.ds(_C + 16 * t, 16)]
            dst_v[b][sl] = d
            dblk_v[b][sl] = jnp.right_shift(d, 7) + _N
            dmod_v[b][sl] = jnp.bitwise_and(d, 127)

        def edge_body(e, carry):
            sacc = jnp.zeros((16,), jnp.float32)
            hsum = []
            for j in range(8):
                sl = pl.ds(16 * j, 16)
                hs_j = hs_r[b][e, sl] + relp_r[b][e, sl]
                z_j = hs_j + hd_r[b][e, sl]
                lr_j = jnp.maximum(z_j, 0.2 * z_j)
                sacc = sacc + lr_j * a_v[sl]
                hsum.append(hs_j)
            for shift in (1, 2, 4, 8):
                perm = jnp.bitwise_xor(lanes, shift)
                sacc = sacc + sacc.at[perm].get(mode="promise_in_bounds")
            ex = jnp.exp(sacc)
            w = dmod_v[b][pl.ds(e, 16)]
            dm = w.at[jnp.zeros((16,), jnp.int32)].get(mode="promise_in_bounds")
            for j in range(8):
                sl = pl.ds(16 * j, 16)
                hs_r[b][e, sl] = ex * hsum[j]
                hit = (lanes + (16 * j)) == dm
                hd_r[b][e, sl] = jnp.where(hit, ex, 0.0)
            return carry

        lax.fori_loop(0, _C, edge_body, 0)

    row0 = wid * _NCH
    pltpu.sync_copy(idx_hbm.at[row0], idx_v[0])
    issue_gathers(0, 0)
    issue_idx(1, row0 + 1)

    def pair_body(g, carry):
        for b in (0, 1):
            bb = 1 - b
            ci = 2 * g + b
            wait_gathers(b)
            compute_chunk(b)
            issue_scatters(b)

            if b == 0:
                @pl.when(g > 0)
                def _():
                    wait_scatters(bb)
            else:
                wait_scatters(bb)

            @pl.when(ci + 1 < _NCH)
            def _():
                wait_idx(bb, row0 + ci + 1)
                issue_gathers(bb, ci + 1)

            @pl.when(ci + 2 < _NCH)
            def _():
                issue_idx(b, row0 + ci + 2)
        return carry

    lax.fori_loop(0, _NG, pair_body, 0)
    wait_scatters(1)
    plsc.subcore_barrier()

    @pl.when(sid == 0)
    def _():
        pltpu.sync_copy(acc, out_hbm.at[cid])


def _sc_edge(hs, hd, relp, idx_packed, a_vec, zeros_acc):
    mesh = plsc.VectorSubcoreMesh(core_axis_name="c", subcore_axis_name="s")
    kern = functools.partial(
        pl.kernel,
        mesh=mesh,
        out_type=jax.ShapeDtypeStruct((2, _ACC_R, _F), jnp.float32),
        scratch_types=[
            pltpu.VMEM((128,), jnp.int32),
            pltpu.VMEM((128,), jnp.int32),
            pltpu.VMEM((_CD,), jnp.int32),
            pltpu.VMEM((_CD,), jnp.int32),
            pltpu.VMEM((_CD,), jnp.int32),
            pltpu.VMEM((_CD,), jnp.int32),
            pltpu.VMEM((_CD + 16,), jnp.int32),
            pltpu.VMEM((_CD + 16,), jnp.int32),
            pltpu.VMEM((_CD, _F), jnp.float32),
            pltpu.VMEM((_CD, _F), jnp.float32),
            pltpu.VMEM((_CD, _F), jnp.float32),
            pltpu.VMEM((_CD, _F), jnp.float32),
            pltpu.VMEM((_C, _F), jnp.float32),
            pltpu.VMEM((_C, _F), jnp.float32),
            pltpu.VMEM((_F,), jnp.float32),
            pltpu.VMEM_SHARED((_ACC_R, _F), jnp.float32),
            pltpu.SemaphoreType.DMA,
            pltpu.SemaphoreType.DMA,
            pltpu.SemaphoreType.DMA,
            pltpu.SemaphoreType.DMA,
            pltpu.SemaphoreType.DMA,
            pltpu.SemaphoreType.DMA,
        ],
    )(_sc_edge_body)
    return kern(hs, hd, relp, idx_packed, a_vec, zeros_acc)


def _mid_body(x_ref, numa_ref, numb_ref, dena_ref, denb_ref, ws_ref, wd_ref,
              inj_ref, hs_ref, hd_ref):
    num = numa_ref[...] + numb_ref[...]
    den = dena_ref[...] + denb_ref[...]
    inj = x_ref[...] + num / (den + 1e-16)
    inj_ref[...] = inj
    hs_ref[...] = jnp.dot(inj, ws_ref[...], preferred_element_type=jnp.float32)
    hd_ref[...] = jnp.dot(inj, wd_ref[...], preferred_element_type=jnp.float32)


def _tc_mid(x, num_a, num_b, den_a, den_b, Ws2, Wd2):
    blk = 1000
    grid = _N // blk
    row_spec = pl.BlockSpec((blk, _F), lambda i: (i, 0))
    den_spec = pl.BlockSpec((blk, 1), lambda i: (i, 0))
    full = lambda s: pl.BlockSpec(s, lambda i: tuple(0 for _ in s))
    return pl.pallas_call(
        _mid_body,
        grid=(grid,),
        in_specs=[row_spec, row_spec, row_spec, den_spec, den_spec,
                  full((_F, _F)), full((_F, _F))],
        out_specs=[row_spec, row_spec, row_spec],
        out_shape=[jax.ShapeDtypeStruct((_N, _F), jnp.float32)] * 3,
    )(x, num_a, num_b, den_a, den_b, Ws2, Wd2)


def _final_body(inj_ref, numa_ref, numb_ref, dena_ref, denb_ref, batch_ref,
                wg_ref, bg_ref, out_ref):
    num = numa_ref[...] + numb_ref[...]
    den = dena_ref[...] + denb_ref[...]
    emb = inj_ref[...] + num / (den + 1e-16)
    gate = jnp.dot(emb, wg_ref[...], preferred_element_type=jnp.float32) \
        + bg_ref[...]
    onehot = (batch_ref[...] == lax.broadcasted_iota(
        jnp.int32, (_N, _B), 1)).astype(jnp.float32)
    ohb = onehot > 0.5
    ms = []
    for k in range(_K):
        gk = gate[:, k:k + 1]
        mk = jnp.max(jnp.where(ohb, gk, -1e30), axis=0, keepdims=True)
        ms.append(mk)
    m_all = jnp.concatenate(ms, axis=0)
    m_node = lax.dot_general(onehot, m_all,
                             (((1,), (1,)), ((), ())),
                             preferred_element_type=jnp.float32)
    ex = jnp.exp(gate - m_node)
    den_b = lax.dot_general(onehot, ex,
                            (((0,), (0,)), ((), ())),
                            preferred_element_type=jnp.float32)
    den_node = jnp.dot(onehot, den_b, preferred_element_type=jnp.float32)
    alpha = ex / (den_node + 1e-16)
    for k in range(_K):
        w = onehot * alpha[:, k:k + 1]
        out_k = lax.dot_general(w, emb,
                                (((0,), (0,)), ((), ())),
                                preferred_element_type=jnp.float32)
        out_ref[:, k, :] = out_k


def _tc_final(inj, num_a, num_b, den_a, den_b, batch2d, W_gate, b_gate2d):
    full = lambda s: pl.BlockSpec(s, lambda: tuple(0 for _ in s))
    return pl.pallas_call(
        _final_body,
        in_specs=[full((_N, _F)), full((_N, _F)), full((_N, _F)),
                  full((_N, 1)), full((_N, 1)),
                  full((_N, 1)), full((_F, _K)), full((1, _K))],
        out_specs=full((_B, _K, _F)),
        out_shape=jax.ShapeDtypeStruct((_B, _K, _F), jnp.float32),
    )(inj, num_a, num_b, den_a, den_b, batch2d, W_gate, b_gate2d)


def kernel(queries, entities, relations, x_coo, batch, W_q, W_inj_src,
           W_inj_dst, a_inj, W_rel_inj, W_enc_src, W_enc_dst, a_enc,
           W_rel_enc, W_gate, b_gate):
    nrows = _E // _C
    idx_packed = jnp.pad(
        x_coo[:, jnp.array([0, 2, 1])]
        .reshape(nrows, _C, 3).transpose(0, 2, 1).reshape(nrows, 3 * _C),
        ((0, 0), (0, 128 - 3 * _C)))
    batch2d = batch.reshape(_N, 1)
    zeros_acc = jnp.zeros((_ACC_R, _F), jnp.float32)
    b_gate2d = b_gate.reshape(1, _K)

    def split_acc(acc):
        num_a = acc[0, :_N, :]
        num_b = acc[1, :_N, :]
        den_a = acc[0, _N:_N + _DEN_R, :].reshape(_DEN_R * _F)[:_N].reshape(_N, 1)
        den_b = acc[1, _N:_N + _DEN_R, :].reshape(_DEN_R * _F)[:_N].reshape(_N, 1)
        return num_a, num_b, den_a, den_b

    x, hs1, hd1, relp1, relp2 = _tc_prep(
        entities, queries, batch2d, relations,
        W_q, W_inj_src, W_inj_dst, W_rel_inj, W_rel_enc)

    acc1 = _sc_edge(hs1, hd1, relp1, idx_packed, a_inj, zeros_acc)

    inj, hs2, hd2 = _tc_mid(x, *split_acc(acc1), W_enc_src, W_enc_dst)

    acc2 = _sc_edge(hs2, hd2, relp2, idx_packed, a_enc, zeros_acc)

    return _tc_final(inj, *split_acc(acc2), batch2d, W_gate, b_gate2d)


# speedup vs baseline: 1.6157x; 1.2692x over previous
import functools

import jax
import jax.numpy as jnp
from jax import lax
from jax.experimental import pallas as pl
from jax.experimental.pallas import tpu as pltpu
from jax.experimental.pallas import tpu_sc as plsc

_N = 10000
_E = 320000
_F = 128
_B = 16
_K = 3
_R = 10000

_NW = 32
_EPW = _E // _NW
_C = 40
_CD = 48
_NCH = _EPW // _C
_NG = _NCH // 2
_DEN_R = (_N + 127) // 128
_DEN_RP = 80
_ACC_R = _N + _DEN_RP


def _prep_body(ent_ref, q_ref, batch_ref, rel_ref, wq_ref, ws1_ref, wd1_ref,
               wr1_ref, wr2_ref, x_ref, hs1_ref, hd1_ref, relp1_ref, relp2_ref):
    qw = jnp.dot(q_ref[...], wq_ref[...], preferred_element_type=jnp.float32)
    onehot = (batch_ref[...] == lax.broadcasted_iota(
        jnp.int32, (batch_ref.shape[0], _B), 1)).astype(jnp.float32)
    x = ent_ref[...] + jnp.dot(onehot, qw, preferred_element_type=jnp.float32)
    x_ref[...] = x
    hs1_ref[...] = jnp.dot(x, ws1_ref[...], preferred_element_type=jnp.float32)
    hd1_ref[...] = jnp.dot(x, wd1_ref[...], preferred_element_type=jnp.float32)
    r = rel_ref[...]
    relp1_ref[...] = jnp.dot(r, wr1_ref[...], preferred_element_type=jnp.float32)
    relp2_ref[...] = jnp.dot(r, wr2_ref[...], preferred_element_type=jnp.float32)


def _tc_prep(entities, queries, batch2d, relations, W_q, Ws1, Wd1, Wr1, Wr2):
    blk = 1000
    grid = _N // blk
    row_spec = pl.BlockSpec((blk, _F), lambda i: (i, 0))
    full = lambda s: pl.BlockSpec(s, lambda i: tuple(0 for _ in s))
    return pl.pallas_call(
        _prep_body,
        grid=(grid,),
        in_specs=[
            row_spec,
            full((_B, _F)),
            pl.BlockSpec((blk, 1), lambda i: (i, 0)),
            row_spec,
            full((_F, _F)), full((_F, _F)), full((_F, _F)),
            full((_F, _F)), full((_F, _F)),
        ],
        out_specs=[row_spec, row_spec, row_spec, row_spec, row_spec],
        out_shape=[jax.ShapeDtypeStruct((_N, _F), jnp.float32)] * 3
        + [jax.ShapeDtypeStruct((_R, _F), jnp.float32)] * 2,
    )(entities, queries, batch2d, relations, W_q, Ws1, Wd1, Wr1, Wr2)


def _sc_edge_body(hs_hbm, hd_hbm, relp_hbm, idx_hbm, a_hbm, z_hbm, out_hbm,
                  idx_v0, idx_v1, dst_v0, dst_v1, dblk_v0, dblk_v1, dmod_v0,
                  dmod_v1, hs_r0, hs_r1, hd_r0, hd_r1, relp_r0, relp_r1, a_v,
                  acc, g0, g1, i0, i1, sc0, sc1):
    cid = lax.axis_index("c")
    sid = lax.axis_index("s")
    wid = cid * 16 + sid

    idx_v = (idx_v0, idx_v1)
    dst_v = (dst_v0, dst_v1)
    dblk_v = (dblk_v0, dblk_v1)
    dmod_v = (dmod_v0, dmod_v1)
    hs_r = (hs_r0, hs_r1)
    hd_r = (hd_r0, hd_r1)
    relp_r = (relp_r0, relp_r1)
    gsem = (g0, g1)
    isem = (i0, i1)
    ssem = (sc0, sc1)

    @pl.when(sid == 0)
    def _():
        pltpu.sync_copy(z_hbm, acc)

    pltpu.sync_copy(a_hbm, a_v)
    plsc.subcore_barrier()

    lanes = lax.broadcasted_iota(jnp.int32, (16,), 0)
    zf = jnp.zeros((16,), jnp.float32)

    for b in (0, 1):
        for r in range(_C, _CD):
            for j in range(8):
                hs_r[b][r, pl.ds(16 * j, 16)] = zf
                hd_r[b][r, pl.ds(16 * j, 16)] = zf

    def issue_gathers(b, ci):
        pltpu.async_copy(hs_hbm.at[idx_v[b].at[pl.ds(0, _C)]],
                         hs_r[b].at[pl.ds(0, _C)], gsem[b])
        pltpu.async_copy(hd_hbm.at[idx_v[b].at[pl.ds(_C, _C)]],
                         hd_r[b].at[pl.ds(0, _C)], gsem[b])
        pltpu.async_copy(relp_hbm.at[idx_v[b].at[pl.ds(2 * _C, _C)]],
                         relp_r[b], gsem[b])

    def wait_gathers(b):
        pltpu.make_async_copy(hs_hbm.at[idx_v[b].at[pl.ds(0, _C)]],
                              hs_r[b].at[pl.ds(0, _C)], gsem[b]).wait()
        pltpu.make_async_copy(hd_hbm.at[idx_v[b].at[pl.ds(_C, _C)]],
                              hd_r[b].at[pl.ds(0, _C)], gsem[b]).wait()
        pltpu.make_async_copy(relp_hbm.at[idx_v[b].at[pl.ds(2 * _C, _C)]],
                              relp_r[b], gsem[b]).wait()

    def issue_idx(b, row):
        pltpu.async_copy(idx_hbm.at[row], idx_v[b], isem[b])

    def wait_idx(b, row):
        pltpu.make_async_copy(idx_hbm.at[row], idx_v[b], isem[b]).wait()

    def issue_scatters(b):
        pltpu.async_copy(hs_r[b], acc.at[dst_v[b]], ssem[b], add=True)
        pltpu.async_copy(hd_r[b], acc.at[dblk_v[b]], ssem[b], add=True)

    def wait_scatters(b):
        pltpu.make_async_copy(hs_r[b], acc.at[dst_v[b]], ssem[b]).wait()
        pltpu.make_async_copy(hd_r[b], acc.at[dblk_v[b]], ssem[b]).wait()

    def compute_chunk(b):
        for t in range(_CD // 16):
            sl = pl.ds(16 * t, 16)
            d = idx_v[b][pl.ds(_C + 16 * t, 16)]
            dst_v[b][sl] = d
            dblk_v[b][sl] = jnp.right_shift(d, 7) + _N
            dmod_v[b][sl] = jnp.bitwise_and(d, 127)

        @plsc.parallel_loop(0, _C, 1, unroll=4)
        def edge_body(e):
            sacc = jnp.zeros((16,), jnp.float32)
            hsum = []
            for j in range(8):
                sl = pl.ds(16 * j, 16)
                hs_j = hs_r[b][e, sl] + relp_r[b][e, sl]
                z_j = hs_j + hd_r[b][e, sl]
                lr_j = jnp.maximum(z_j, 0.2 * z_j)
                sacc = sacc + lr_j * a_v[sl]
                hsum.append(hs_j)
            for shift in (1, 2, 4, 8):
                perm = jnp.bitwise_xor(lanes, shift)
                sacc = sacc + sacc.at[perm].get(mode="promise_in_bounds")
            ex = jnp.exp(sacc)
            w = dmod_v[b][pl.ds(e, 16)]
            dm = w.at[jnp.zeros((16,), jnp.int32)].get(mode="promise_in_bounds")
            for j in range(8):
                sl = pl.ds(16 * j, 16)
                hs_r[b][e, sl] = ex * hsum[j]
                hit = (lanes + (16 * j)) == dm
                hd_r[b][e, sl] = jnp.where(hit, ex, 0.0)

    row0 = wid * _NCH
    pltpu.sync_copy(idx_hbm.at[row0], idx_v[0])
    issue_gathers(0, 0)
    issue_idx(1, row0 + 1)

    def pair_body(g, carry):
        for b in (0, 1):
            bb = 1 - b
            ci = 2 * g + b
            wait_gathers(b)
            compute_chunk(b)
            issue_scatters(b)

            if b == 0:
                @pl.when(g > 0)
                def _():
                    wait_scatters(bb)
            else:
                wait_scatters(bb)

            @pl.when(ci + 1 < _NCH)
            def _():
                wait_idx(bb, row0 + ci + 1)
                issue_gathers(bb, ci + 1)

            @pl.when(ci + 2 < _NCH)
            def _():
                issue_idx(b, row0 + ci + 2)
        return carry

    lax.fori_loop(0, _NG, pair_body, 0)
    wait_scatters(1)
    plsc.subcore_barrier()

    @pl.when(sid == 0)
    def _():
        pltpu.sync_copy(acc, out_hbm.at[cid])


def _sc_edge(hs, hd, relp, idx_packed, a_vec, zeros_acc):
    mesh = plsc.VectorSubcoreMesh(core_axis_name="c", subcore_axis_name="s")
    kern = functools.partial(
        pl.kernel,
        mesh=mesh,
        out_type=jax.ShapeDtypeStruct((2, _ACC_R, _F), jnp.float32),
        scratch_types=[
            pltpu.VMEM((128,), jnp.int32),
            pltpu.VMEM((128,), jnp.int32),
            pltpu.VMEM((_CD,), jnp.int32),
            pltpu.VMEM((_CD,), jnp.int32),
            pltpu.VMEM((_CD,), jnp.int32),
            pltpu.VMEM((_CD,), jnp.int32),
            pltpu.VMEM((_CD + 16,), jnp.int32),
            pltpu.VMEM((_CD + 16,), jnp.int32),
            pltpu.VMEM((_CD, _F), jnp.float32),
            pltpu.VMEM((_CD, _F), jnp.float32),
            pltpu.VMEM((_CD, _F), jnp.float32),
            pltpu.VMEM((_CD, _F), jnp.float32),
            pltpu.VMEM((_C, _F), jnp.float32),
            pltpu.VMEM((_C, _F), jnp.float32),
            pltpu.VMEM((_F,), jnp.float32),
            pltpu.VMEM_SHARED((_ACC_R, _F), jnp.float32),
            pltpu.SemaphoreType.DMA,
            pltpu.SemaphoreType.DMA,
            pltpu.SemaphoreType.DMA,
            pltpu.SemaphoreType.DMA,
            pltpu.SemaphoreType.DMA,
            pltpu.SemaphoreType.DMA,
        ],
    )(_sc_edge_body)
    return kern(hs, hd, relp, idx_packed, a_vec, zeros_acc)


def _mid_body(x_ref, numa_ref, numb_ref, dena_ref, denb_ref, ws_ref, wd_ref,
              inj_ref, hs_ref, hd_ref):
    num = numa_ref[...] + numb_ref[...]
    den = dena_ref[...] + denb_ref[...]
    inj = x_ref[...] + num / (den + 1e-16)
    inj_ref[...] = inj
    hs_ref[...] = jnp.dot(inj, ws_ref[...], preferred_element_type=jnp.float32)
    hd_ref[...] = jnp.dot(inj, wd_ref[...], preferred_element_type=jnp.float32)


def _tc_mid(x, num_a, num_b, den_a, den_b, Ws2, Wd2):
    blk = 1000
    grid = _N // blk
    row_spec = pl.BlockSpec((blk, _F), lambda i: (i, 0))
    den_spec = pl.BlockSpec((blk, 1), lambda i: (i, 0))
    full = lambda s: pl.BlockSpec(s, lambda i: tuple(0 for _ in s))
    return pl.pallas_call(
        _mid_body,
        grid=(grid,),
        in_specs=[row_spec, row_spec, row_spec, den_spec, den_spec,
                  full((_F, _F)), full((_F, _F))],
        out_specs=[row_spec, row_spec, row_spec],
        out_shape=[jax.ShapeDtypeStruct((_N, _F), jnp.float32)] * 3,
    )(x, num_a, num_b, den_a, den_b, Ws2, Wd2)


def _final_body(inj_ref, numa_ref, numb_ref, dena_ref, denb_ref, batch_ref,
                wg_ref, bg_ref, out_ref):
    num = numa_ref[...] + numb_ref[...]
    den = dena_ref[...] + denb_ref[...]
    emb = inj_ref[...] + num / (den + 1e-16)
    gate = jnp.dot(emb, wg_ref[...], preferred_element_type=jnp.float32) \
        + bg_ref[...]
    onehot = (batch_ref[...] == lax.broadcasted_iota(
        jnp.int32, (_N, _B), 1)).astype(jnp.float32)
    ohb = onehot > 0.5
    ms = []
    for k in range(_K):
        gk = gate[:, k:k + 1]
        mk = jnp.max(jnp.where(ohb, gk, -1e30), axis=0, keepdims=True)
        ms.append(mk)
    m_all = jnp.concatenate(ms, axis=0)
    m_node = lax.dot_general(onehot, m_all,
                             (((1,), (1,)), ((), ())),
                             preferred_element_type=jnp.float32)
    ex = jnp.exp(gate - m_node)
    den_b = lax.dot_general(onehot, ex,
                            (((0,), (0,)), ((), ())),
                            preferred_element_type=jnp.float32)
    den_node = jnp.dot(onehot, den_b, preferred_element_type=jnp.float32)
    alpha = ex / (den_node + 1e-16)
    for k in range(_K):
        w = onehot * alpha[:, k:k + 1]
        out_k = lax.dot_general(w, emb,
                                (((0,), (0,)), ((), ())),
                                preferred_element_type=jnp.float32)
        out_ref[:, k, :] = out_k


def _tc_final(inj, num_a, num_b, den_a, den_b, batch2d, W_gate, b_gate2d):
    full = lambda s: pl.BlockSpec(s, lambda: tuple(0 for _ in s))
    return pl.pallas_call(
        _final_body,
        in_specs=[full((_N, _F)), full((_N, _F)), full((_N, _F)),
                  full((_N, 1)), full((_N, 1)),
                  full((_N, 1)), full((_F, _K)), full((1, _K))],
        out_specs=full((_B, _K, _F)),
        out_shape=jax.ShapeDtypeStruct((_B, _K, _F), jnp.float32),
    )(inj, num_a, num_b, den_a, den_b, batch2d, W_gate, b_gate2d)


def kernel(queries, entities, relations, x_coo, batch, W_q, W_inj_src,
           W_inj_dst, a_inj, W_rel_inj, W_enc_src, W_enc_dst, a_enc,
           W_rel_enc, W_gate, b_gate):
    nrows = _E // _C
    idx_packed = jnp.pad(
        x_coo[:, jnp.array([0, 2, 1])]
        .reshape(nrows, _C, 3).transpose(0, 2, 1).reshape(nrows, 3 * _C),
        ((0, 0), (0, 128 - 3 * _C)))
    batch2d = batch.reshape(_N, 1)
    zeros_acc = jnp.zeros((_ACC_R, _F), jnp.float32)
    b_gate2d = b_gate.reshape(1, _K)

    def split_acc(acc):
        num_a = acc[0, :_N, :]
        num_b = acc[1, :_N, :]
        den_a = acc[0, _N:_N + _DEN_R, :].reshape(_DEN_R * _F)[:_N].reshape(_N, 1)
        den_b = acc[1, _N:_N + _DEN_R, :].reshape(_DEN_R * _F)[:_N].reshape(_N, 1)
        return num_a, num_b, den_a, den_b

    x, hs1, hd1, relp1, relp2 = _tc_prep(
        entities, queries, batch2d, relations,
        W_q, W_inj_src, W_inj_dst, W_rel_inj, W_rel_enc)

    acc1 = _sc_edge(hs1, hd1, relp1, idx_packed, a_inj, zeros_acc)

    inj, hs2, hd2 = _tc_mid(x, *split_acc(acc1), W_enc_src, W_enc_dst)

    acc2 = _sc_edge(hs2, hd2, relp2, idx_packed, a_enc, zeros_acc)

    return _tc_final(inj, *split_acc(acc2), batch2d, W_gate, b_gate2d)
